# baseline (speedup 1.0000x reference)
import functools

import jax
import jax.numpy as jnp
from jax import lax
from jax.experimental import pallas as pl
from jax.experimental.pallas import tpu as pltpu
from jax.experimental.pallas import tpu_sc as plsc

VOCAB = 50257
NUM_USERS = 100000
NUM_ITEMS = 100000
N_EMBD = 768
VU = VOCAB + NUM_USERS

NC = 2
NS = 16
NW = NC * NS
LANES = 16

TOTAL = 1024 * 50
CHUNK = TOTAL // NW
G = 64
LIST = CHUNK + G


def _permute(x, idx):
    dnums = lax.GatherDimensionNumbers(
        offset_dims=(), collapsed_slice_dims=(0,), start_index_map=(0,))
    return lax.gather(x, idx[:, None], dnums, slice_sizes=(1,),
                      mode=lax.GatherScatterMode.PROMISE_IN_BOUNDS)


def _prefix_sum(x, lane):
    r = x
    for sh in (1, 2, 4, 8):
        prev = _permute(r, jnp.maximum(lane - sh, 0))
        r = r + jnp.where(lane >= sh, prev, 0)
    return r


def _compact_src(r, lane):
    tgt = lane + 1
    lo = jnp.zeros((LANES,), jnp.int32)
    for sh in (8, 4, 2, 1):
        probe = jnp.minimum(lo + (sh - 1), LANES - 1)
        val = _permute(r, probe)
        lo = lo + jnp.where(val < tgt, sh, 0)
    return jnp.minimum(lo, LANES - 1)


def _body(ids_hbm, wte_hbm, user_hbm, item_hbm, out_hbm,
          ids_v, idx0, idx1, idx2, pos0, pos1, pos2,
          idx_stage, pos_stage, rows_a, rows_b,
          sem_g0, sem_g1, sem_s0, sem_s1):
    wid = lax.axis_index("s") * NC + lax.axis_index("c")
    base = wid * CHUNK
    idx_lists = (idx0, idx1, idx2)
    pos_lists = (pos0, pos1, pos2)

    pltpu.sync_copy(ids_hbm.at[pl.ds(base, CHUNK)], ids_v)

    lane = lax.iota(jnp.int32, LANES)
    zero_lane = jnp.zeros((LANES,), jnp.int32)

    def compact(g, carry):
        o0, o1, o2, f0i, f0p, f1i, f1p, f2i, f2p = carry
        v = ids_v[pl.ds(g * LANES, LANES)]
        pos = lane + (g * LANES + base)
        m0 = v < VOCAB
        m2 = v >= VU
        r0 = _prefix_sum(jnp.where(m0, 1, 0), lane)
        r2 = _prefix_sum(jnp.where(m2, 1, 0), lane)
        r1 = (lane + 1) - r0 - r2
        n0 = r0[LANES - 1]
        n2 = r2[LANES - 1]
        cnts = (n0, LANES - n0 - n2, n2)
        firsts = []
        for r, o, cnt, fi, fp, idx_l, pos_l, off in (
                (r0, o0, cnts[0], f0i, f0p, idx0, pos0, 0),
                (r1, o1, cnts[1], f1i, f1p, idx1, pos1, VOCAB),
                (r2, o2, cnts[2], f2i, f2p, idx2, pos2, VU)):
            src = _compact_src(r, lane)
            cv = _permute(v, src) - off
            cp = _permute(pos, src)
            idx_l[pl.ds(o, LANES)] = cv
            pos_l[pl.ds(o, LANES)] = cp
            is_first = jnp.logical_and(o == 0, cnt > 0)
            fi = jnp.where(is_first, _permute(cv, zero_lane), fi)
            fp = jnp.where(is_first, _permute(cp, zero_lane), fp)
            firsts += [fi, fp]
        return (o0 + cnts[0], o1 + cnts[1], o2 + cnts[2], *firsts)

    z = jnp.int32(0)
    c0, c1, c2, f0i, f0p, f1i, f1p, f2i, f2p = lax.fori_loop(
        0, CHUNK // LANES, compact,
        (z, z, z, zero_lane, zero_lane, zero_lane, zero_lane, zero_lane,
         zero_lane))

    for c, fi, fp, idx_l, pos_l in ((c0, f0i, f0p, idx0, pos0),
                                    (c1, f1i, f1p, idx1, pos1),
                                    (c2, f2i, f2p, idx2, pos2)):
        for k in range(G // LANES):
            idx_l[pl.ds(c + k * LANES, LANES)] = fi
            pos_l[pl.ds(c + k * LANES, LANES)] = fp

    rows = (rows_a, rows_b)
    sg = (sem_g0, sem_g1)
    ss = (sem_s0, sem_s1)

    for t, (table, count) in enumerate(((wte_hbm, c0), (user_hbm, c1),
                                        (item_hbm, c2))):
        n_groups = (count + (G - 1)) // G

        def stage_and_gather(q, b, t=t, table=table):
            for k in range(G // LANES):
                idx_stage.at[b][pl.ds(k * LANES, LANES)] = (
                    idx_lists[t][pl.ds(q * G + k * LANES, LANES)])
                pos_stage.at[b][pl.ds(k * LANES, LANES)] = (
                    pos_lists[t][pl.ds(q * G + k * LANES, LANES)])
            pltpu.async_copy(table.at[idx_stage.at[b]], rows[b], sg[b])

        def wait_and_scatter(q, b, t=t, table=table):
            pltpu.make_async_copy(table.at[idx_stage.at[b]], rows[b],
                                  sg[b]).wait()
            pltpu.async_copy(rows[b], out_hbm.at[pos_stage.at[b]], ss[b])

        def drain_scatter(b):
            pltpu.make_async_copy(rows[b], out_hbm.at[pos_stage.at[b]],
                                  ss[b]).wait()

        def pair(p, _):
            for b in range(2):
                q = 2 * p + b

                @pl.when(q < n_groups)
                def _(q=q, b=b):
                    @pl.when(q >= 2)
                    def _():
                        drain_scatter(b)
                    stage_and_gather(q, b)
            for b in range(2):
                q = 2 * p + b

                @pl.when(q < n_groups)
                def _(q=q, b=b):
                    wait_and_scatter(q, b)
            return 0

        lax.fori_loop(0, (n_groups + 1) // 2, pair, 0)

        @pl.when(n_groups >= 2)
        def _():
            drain_scatter(0)
            drain_scatter(1)

        @pl.when(n_groups == 1)
        def _():
            drain_scatter(0)


@jax.jit
def _lookup(ids_flat, wte, user_embeddings, item_embeddings):
    mesh = plsc.VectorSubcoreMesh(core_axis_name="c", subcore_axis_name="s")
    return pl.kernel(
        _body,
        out_type=jax.ShapeDtypeStruct((TOTAL, N_EMBD), jnp.float32),
        mesh=mesh,
        scratch_types=[
            pltpu.VMEM((CHUNK,), jnp.int32),
            pltpu.VMEM((LIST,), jnp.int32),
            pltpu.VMEM((LIST,), jnp.int32),
            pltpu.VMEM((LIST,), jnp.int32),
            pltpu.VMEM((LIST,), jnp.int32),
            pltpu.VMEM((LIST,), jnp.int32),
            pltpu.VMEM((LIST,), jnp.int32),
            pltpu.VMEM((2, G), jnp.int32),
            pltpu.VMEM((2, G), jnp.int32),
            pltpu.VMEM((G, N_EMBD), jnp.float32),
            pltpu.VMEM((G, N_EMBD), jnp.float32),
            pltpu.SemaphoreType.DMA,
            pltpu.SemaphoreType.DMA,
            pltpu.SemaphoreType.DMA,
            pltpu.SemaphoreType.DMA,
        ],
    )(ids_flat, wte, user_embeddings, item_embeddings)


def _relayout_body(in_ref, out_ref):
    out_ref[...] = in_ref[...].reshape(out_ref.shape)


_BB = 2


@jax.jit
def _relayout(x):
    rows_per_block = _BB * 50 * N_EMBD // 128
    return pl.pallas_call(
        _relayout_body,
        grid=(1024 // _BB,),
        in_specs=[pl.BlockSpec((rows_per_block, 128), lambda b: (b, 0))],
        out_specs=pl.BlockSpec((_BB, 50, N_EMBD), lambda b: (b, 0, 0)),
        out_shape=jax.ShapeDtypeStruct((1024, 50, N_EMBD), jnp.float32),
    )(x)


def kernel(input_ids, wte, user_embeddings, item_embeddings):
    ids_flat = input_ids.reshape(-1).astype(jnp.int32)
    out = _lookup(ids_flat, wte, user_embeddings, item_embeddings)
    return _relayout(out.reshape(TOTAL * N_EMBD // 128, 128))


# speedup vs baseline: 5.1984x; 5.1984x over previous
import functools

import jax
import jax.numpy as jnp
from jax import lax
from jax.experimental import pallas as pl
from jax.experimental.pallas import tpu as pltpu
from jax.experimental.pallas import tpu_sc as plsc

VOCAB = 50257
NUM_USERS = 100000
NUM_ITEMS = 100000
N_EMBD = 768
VU = VOCAB + NUM_USERS

NC = 2
NS = 16
NW = NC * NS
LANES = 16

TOTAL = 1024 * 50
CHUNK = TOTAL // NW
G = 32
NB = 4
LIST = CHUNK + G


def _permute(x, idx):
    dnums = lax.GatherDimensionNumbers(
        offset_dims=(), collapsed_slice_dims=(0,), start_index_map=(0,))
    return lax.gather(x, idx[:, None], dnums, slice_sizes=(1,),
                      mode=lax.GatherScatterMode.PROMISE_IN_BOUNDS)


def _prefix_sum(x, lane):
    r = x
    for sh in (1, 2, 4, 8):
        prev = _permute(r, jnp.maximum(lane - sh, 0))
        r = r + jnp.where(lane >= sh, prev, 0)
    return r


def _compact_src(r, lane):
    tgt = lane + 1
    lo = jnp.zeros((LANES,), jnp.int32)
    for sh in (8, 4, 2, 1):
        probe = jnp.minimum(lo + (sh - 1), LANES - 1)
        val = _permute(r, probe)
        lo = lo + jnp.where(val < tgt, sh, 0)
    return jnp.minimum(lo, LANES - 1)


def _body(ids_hbm, wte_hbm, user_hbm, item_hbm, out_hbm,
          ids_v, idx0, idx1, idx2, pos0, pos1, pos2,
          idx_stage, pos_stage, rows_a, rows_b, rows_c, rows_d,
          sem_g0, sem_g1, sem_g2, sem_g3,
          sem_s0, sem_s1, sem_s2, sem_s3):
    wid = lax.axis_index("s") * NC + lax.axis_index("c")
    base = wid * CHUNK
    idx_lists = (idx0, idx1, idx2)
    pos_lists = (pos0, pos1, pos2)

    pltpu.sync_copy(ids_hbm.at[pl.ds(base, CHUNK)], ids_v)

    lane = lax.iota(jnp.int32, LANES)
    zero_lane = jnp.zeros((LANES,), jnp.int32)

    def compact(g, carry):
        o0, o1, o2, f0i, f0p, f1i, f1p, f2i, f2p = carry
        v = ids_v[pl.ds(g * LANES, LANES)]
        i = lane + g * LANES
        bb = lax.shift_right_logical(i * 1311, 16)
        pos = (i - bb * 50) * 1024 + (bb + (base // 50))
        m0 = v < VOCAB
        m2 = v >= VU
        r0 = _prefix_sum(jnp.where(m0, 1, 0), lane)
        r2 = _prefix_sum(jnp.where(m2, 1, 0), lane)
        r1 = (lane + 1) - r0 - r2
        n0 = r0[LANES - 1]
        n2 = r2[LANES - 1]
        cnts = (n0, LANES - n0 - n2, n2)
        firsts = []
        for r, o, cnt, fi, fp, idx_l, pos_l, off in (
                (r0, o0, cnts[0], f0i, f0p, idx0, pos0, 0),
                (r1, o1, cnts[1], f1i, f1p, idx1, pos1, VOCAB),
                (r2, o2, cnts[2], f2i, f2p, idx2, pos2, VU)):
            src = _compact_src(r, lane)
            cv = _permute(v, src) - off
            cp = _permute(pos, src)
            idx_l[pl.ds(o, LANES)] = cv
            pos_l[pl.ds(o, LANES)] = cp
            is_first = jnp.logical_and(o == 0, cnt > 0)
            fi = jnp.where(is_first, _permute(cv, zero_lane), fi)
            fp = jnp.where(is_first, _permute(cp, zero_lane), fp)
            firsts += [fi, fp]
        return (o0 + cnts[0], o1 + cnts[1], o2 + cnts[2], *firsts)

    z = jnp.int32(0)
    c0, c1, c2, f0i, f0p, f1i, f1p, f2i, f2p = lax.fori_loop(
        0, CHUNK // LANES, compact,
        (z, z, z, zero_lane, zero_lane, zero_lane, zero_lane, zero_lane,
         zero_lane))

    for c, fi, fp, idx_l, pos_l in ((c0, f0i, f0p, idx0, pos0),
                                    (c1, f1i, f1p, idx1, pos1),
                                    (c2, f2i, f2p, idx2, pos2)):
        for k in range(G // LANES):
            idx_l[pl.ds(c + k * LANES, LANES)] = fi
            pos_l[pl.ds(c + k * LANES, LANES)] = fp

    rows = (rows_a, rows_b, rows_c, rows_d)
    sg = (sem_g0, sem_g1, sem_g2, sem_g3)
    ss = (sem_s0, sem_s1, sem_s2, sem_s3)

    n0 = (c0 + (G - 1)) // G
    n1 = (c1 + (G - 1)) // G
    n2 = (c2 + (G - 1)) // G
    t1 = n0
    t2 = n0 + n1
    n_all = t2 + n2

    tables = ((wte_hbm, 0), (user_hbm, t1), (item_hbm, t2))
    bounds = ((0, t1), (t1, t2), (t2, n_all))

    def stage_and_gather(q, b):
        for t, (table, lo) in enumerate(tables):
            @pl.when(jnp.logical_and(q >= lo, q < bounds[t][1]))
            def _(t=t, table=table, lo=lo):
                qt = q - lo
                for k in range(G // LANES):
                    idx_stage.at[b][pl.ds(k * LANES, LANES)] = (
                        idx_lists[t][pl.ds(qt * G + k * LANES, LANES)])
                    pos_stage.at[b][pl.ds(k * LANES, LANES)] = (
                        pos_lists[t][pl.ds(qt * G + k * LANES, LANES)])
                pltpu.async_copy(table.at[idx_stage.at[b]], rows[b], sg[b])

    def wait_gather(b):
        pltpu.make_async_copy(wte_hbm.at[idx_stage.at[b]], rows[b],
                              sg[b]).wait()

    def drain_scatter(b):
        pltpu.make_async_copy(rows[b], out_hbm.at[pos_stage.at[b]],
                              ss[b]).wait()

    def ring_round(rd, _):
        for b in range(NB):
            q = rd * NB + b

            @pl.when(q < n_all)
            def _(q=q, b=b):
                @pl.when(q >= NB)
                def _():
                    drain_scatter(b)
                stage_and_gather(q, b)
        for b in range(NB):
            q = rd * NB + b

            @pl.when(q < n_all)
            def _(q=q, b=b):
                wait_gather(b)
                pltpu.async_copy(rows[b], out_hbm.at[pos_stage.at[b]], ss[b])
        return 0

    lax.fori_loop(0, (n_all + NB - 1) // NB, ring_round, 0)

    for b in range(NB):
        @pl.when(b < n_all)
        def _(b=b):
            drain_scatter(b)


@jax.jit
def _lookup(ids_flat, wte, user_embeddings, item_embeddings):
    mesh = plsc.VectorSubcoreMesh(core_axis_name="c", subcore_axis_name="s")
    return pl.kernel(
        _body,
        out_type=jax.ShapeDtypeStruct((TOTAL, N_EMBD), jnp.float32),
        mesh=mesh,
        scratch_types=[
            pltpu.VMEM((CHUNK,), jnp.int32),
            pltpu.VMEM((LIST,), jnp.int32),
            pltpu.VMEM((LIST,), jnp.int32),
            pltpu.VMEM((LIST,), jnp.int32),
            pltpu.VMEM((LIST,), jnp.int32),
            pltpu.VMEM((LIST,), jnp.int32),
            pltpu.VMEM((LIST,), jnp.int32),
            pltpu.VMEM((NB, G), jnp.int32),
            pltpu.VMEM((NB, G), jnp.int32),
            pltpu.VMEM((G, N_EMBD), jnp.float32),
            pltpu.VMEM((G, N_EMBD), jnp.float32),
            pltpu.VMEM((G, N_EMBD), jnp.float32),
            pltpu.VMEM((G, N_EMBD), jnp.float32),
            pltpu.SemaphoreType.DMA,
            pltpu.SemaphoreType.DMA,
            pltpu.SemaphoreType.DMA,
            pltpu.SemaphoreType.DMA,
            pltpu.SemaphoreType.DMA,
            pltpu.SemaphoreType.DMA,
            pltpu.SemaphoreType.DMA,
            pltpu.SemaphoreType.DMA,
        ],
    )(ids_flat, wte, user_embeddings, item_embeddings)


def kernel(input_ids, wte, user_embeddings, item_embeddings):
    ids_flat = input_ids.reshape(-1).astype(jnp.int32)
    out = _lookup(ids_flat, wte, user_embeddings, item_embeddings)
    return out.reshape(50, 1024, N_EMBD).transpose(1, 0, 2)


# speedup vs baseline: 5.6637x; 1.0895x over previous
import functools

import jax
import jax.numpy as jnp
from jax import lax
from jax.experimental import pallas as pl
from jax.experimental.pallas import tpu as pltpu
from jax.experimental.pallas import tpu_sc as plsc

VOCAB = 50257
NUM_USERS = 100000
NUM_ITEMS = 100000
N_EMBD = 768
VU = VOCAB + NUM_USERS

NC = 2
NS = 16
NW = NC * NS
LANES = 16

TOTAL = 1024 * 50
CHUNK = TOTAL // NW
G = 32
NB = 4
LIST = CHUNK + G


def _permute(x, idx):
    dnums = lax.GatherDimensionNumbers(
        offset_dims=(), collapsed_slice_dims=(0,), start_index_map=(0,))
    return lax.gather(x, idx[:, None], dnums, slice_sizes=(1,),
                      mode=lax.GatherScatterMode.PROMISE_IN_BOUNDS)


def _prefix_sum(x, lane):
    r = x
    for sh in (1, 2, 4, 8):
        prev = _permute(r, jnp.maximum(lane - sh, 0))
        r = r + jnp.where(lane >= sh, prev, 0)
    return r


def _compact_src(r, lane):
    tgt = lane + 1
    lo = jnp.zeros((LANES,), jnp.int32)
    for sh in (8, 4, 2, 1):
        probe = jnp.minimum(lo + (sh - 1), LANES - 1)
        val = _permute(r, probe)
        lo = lo + jnp.where(val < tgt, sh, 0)
    return jnp.minimum(lo, LANES - 1)


def _body(ids_hbm, wte_hbm, user_hbm, item_hbm, out_hbm,
          ids_v, idx0, idx1, idx2, pos0, pos1, pos2,
          idx_stage, pos_stage, rows_a, rows_b, rows_c, rows_d,
          sem_g0, sem_g1, sem_g2, sem_g3,
          sem_s0, sem_s1, sem_s2, sem_s3):
    wid = lax.axis_index("s") * NC + lax.axis_index("c")
    base = wid * CHUNK
    idx_lists = (idx0, idx1, idx2)
    pos_lists = (pos0, pos1, pos2)

    pltpu.sync_copy(ids_hbm.at[pl.ds(base, CHUNK)], ids_v)

    lane = lax.iota(jnp.int32, LANES)
    zero_lane = jnp.zeros((LANES,), jnp.int32)

    def compact(g, carry):
        o0, o1, o2, f0i, f0p, f1i, f1p, f2i, f2p = carry
        v = ids_v[pl.ds(g * LANES, LANES)]
        i = lane + g * LANES
        bb = lax.shift_right_logical(i * 1311, 16)
        pos = (i - bb * 50) * 1024 + (bb + (base // 50))
        m0 = v < VOCAB
        m2 = v >= VU
        r0 = _prefix_sum(jnp.where(m0, 1, 0), lane)
        r2 = _prefix_sum(jnp.where(m2, 1, 0), lane)
        r1 = (lane + 1) - r0 - r2
        n0 = r0[LANES - 1]
        n2 = r2[LANES - 1]
        cnts = (n0, LANES - n0 - n2, n2)
        firsts = []
        for r, o, cnt, fi, fp, idx_l, pos_l, off in (
                (r0, o0, cnts[0], f0i, f0p, idx0, pos0, 0),
                (r1, o1, cnts[1], f1i, f1p, idx1, pos1, VOCAB),
                (r2, o2, cnts[2], f2i, f2p, idx2, pos2, VU)):
            src = _compact_src(r, lane)
            cv = _permute(v, src) - off
            cp = _permute(pos, src)
            idx_l[pl.ds(o, LANES)] = cv
            pos_l[pl.ds(o, LANES)] = cp
            is_first = jnp.logical_and(o == 0, cnt > 0)
            fi = jnp.where(is_first, _permute(cv, zero_lane), fi)
            fp = jnp.where(is_first, _permute(cp, zero_lane), fp)
            firsts += [fi, fp]
        return (o0 + cnts[0], o1 + cnts[1], o2 + cnts[2], *firsts)

    z = jnp.int32(0)
    c0, c1, c2, f0i, f0p, f1i, f1p, f2i, f2p = lax.fori_loop(
        0, CHUNK // LANES, compact,
        (z, z, z, zero_lane, zero_lane, zero_lane, zero_lane, zero_lane,
         zero_lane))

    for c, fi, fp, idx_l, pos_l in ((c0, f0i, f0p, idx0, pos0),
                                    (c1, f1i, f1p, idx1, pos1),
                                    (c2, f2i, f2p, idx2, pos2)):
        for k in range(G // LANES):
            idx_l[pl.ds(c + k * LANES, LANES)] = fi
            pos_l[pl.ds(c + k * LANES, LANES)] = fp

    rows = (rows_a, rows_b, rows_c, rows_d)
    sg = (sem_g0, sem_g1, sem_g2, sem_g3)
    ss = (sem_s0, sem_s1, sem_s2, sem_s3)

    n0 = (c0 + (G - 1)) // G
    n1 = (c1 + (G - 1)) // G
    n2 = (c2 + (G - 1)) // G
    t1 = n0
    t2 = n0 + n1
    n_all = t2 + n2

    tables = ((wte_hbm, 0), (user_hbm, t1), (item_hbm, t2))
    bounds = ((0, t1), (t1, t2), (t2, n_all))

    def stage_and_gather(q, b):
        for t, (table, lo) in enumerate(tables):
            @pl.when(jnp.logical_and(q >= lo, q < bounds[t][1]))
            def _(t=t, table=table, lo=lo):
                qt = q - lo
                for k in range(G // LANES):
                    pos_stage.at[b][pl.ds(k * LANES, LANES)] = (
                        pos_lists[t][pl.ds(qt * G + k * LANES, LANES)])
                pltpu.async_copy(
                    table.at[idx_lists[t].at[pl.ds(qt * G, G)]],
                    rows[b], sg[b])

    def wait_gather(b):
        pltpu.make_async_copy(wte_hbm.at[idx0.at[pl.ds(0, G)]], rows[b],
                              sg[b]).wait()

    def drain_scatter(b):
        pltpu.make_async_copy(rows[b], out_hbm.at[pos_stage.at[b]],
                              ss[b]).wait()

    def ring_round(rd, _):
        for b in range(NB):
            q = rd * NB + b

            @pl.when(q < n_all)
            def _(q=q, b=b):
                @pl.when(q >= NB)
                def _():
                    drain_scatter(b)
                stage_and_gather(q, b)
        for b in range(NB):
            q = rd * NB + b

            @pl.when(q < n_all)
            def _(q=q, b=b):
                wait_gather(b)
                pltpu.async_copy(rows[b], out_hbm.at[pos_stage.at[b]], ss[b])
        return 0

    lax.fori_loop(0, (n_all + NB - 1) // NB, ring_round, 0)

    for b in range(NB):
        @pl.when(b < n_all)
        def _(b=b):
            drain_scatter(b)


@jax.jit
def _lookup(ids_flat, wte, user_embeddings, item_embeddings):
    mesh = plsc.VectorSubcoreMesh(core_axis_name="c", subcore_axis_name="s")
    return pl.kernel(
        _body,
        out_type=jax.ShapeDtypeStruct((TOTAL, N_EMBD), jnp.float32),
        mesh=mesh,
        scratch_types=[
            pltpu.VMEM((CHUNK,), jnp.int32),
            pltpu.VMEM((LIST,), jnp.int32),
            pltpu.VMEM((LIST,), jnp.int32),
            pltpu.VMEM((LIST,), jnp.int32),
            pltpu.VMEM((LIST,), jnp.int32),
            pltpu.VMEM((LIST,), jnp.int32),
            pltpu.VMEM((LIST,), jnp.int32),
            pltpu.VMEM((NB, G), jnp.int32),
            pltpu.VMEM((NB, G), jnp.int32),
            pltpu.VMEM((G, N_EMBD), jnp.float32),
            pltpu.VMEM((G, N_EMBD), jnp.float32),
            pltpu.VMEM((G, N_EMBD), jnp.float32),
            pltpu.VMEM((G, N_EMBD), jnp.float32),
            pltpu.SemaphoreType.DMA,
            pltpu.SemaphoreType.DMA,
            pltpu.SemaphoreType.DMA,
            pltpu.SemaphoreType.DMA,
            pltpu.SemaphoreType.DMA,
            pltpu.SemaphoreType.DMA,
            pltpu.SemaphoreType.DMA,
            pltpu.SemaphoreType.DMA,
        ],
    )(ids_flat, wte, user_embeddings, item_embeddings)


def kernel(input_ids, wte, user_embeddings, item_embeddings):
    ids_flat = input_ids.reshape(-1).astype(jnp.int32)
    out = _lookup(ids_flat, wte, user_embeddings, item_embeddings)
    return out.reshape(50, 1024, N_EMBD).transpose(1, 0, 2)


# speedup vs baseline: 5.7157x; 1.0092x over previous
import functools

import jax
import jax.numpy as jnp
from jax import lax
from jax.experimental import pallas as pl
from jax.experimental.pallas import tpu as pltpu
from jax.experimental.pallas import tpu_sc as plsc

VOCAB = 50257
NUM_USERS = 100000
NUM_ITEMS = 100000
N_EMBD = 768
VU = VOCAB + NUM_USERS

NC = 2
NS = 16
NW = NC * NS
LANES = 16

TOTAL = 1024 * 50
CHUNK = TOTAL // NW
G = 32
NB = 4
LIST = CHUNK + G


def _permute(x, idx):
    dnums = lax.GatherDimensionNumbers(
        offset_dims=(), collapsed_slice_dims=(0,), start_index_map=(0,))
    return lax.gather(x, idx[:, None], dnums, slice_sizes=(1,),
                      mode=lax.GatherScatterMode.PROMISE_IN_BOUNDS)


def _prefix_sum(x, lane):
    r = x
    for sh in (1, 2, 4, 8):
        prev = _permute(r, jnp.maximum(lane - sh, 0))
        r = r + jnp.where(lane >= sh, prev, 0)
    return r


def _compact_src(r, lane):
    tgt = lane + 1
    lo = jnp.zeros((LANES,), jnp.int32)
    for sh in (8, 4, 2, 1):
        probe = jnp.minimum(lo + (sh - 1), LANES - 1)
        val = _permute(r, probe)
        lo = lo + jnp.where(val < tgt, sh, 0)
    return jnp.minimum(lo, LANES - 1)


def _body(ids_hbm, wte_hbm, user_hbm, item_hbm, out_hbm,
          ids_v, idx0, idx1, idx2, pos0, pos1, pos2,
          idx_stage, pos_stage, rows_a, rows_b, rows_c, rows_d,
          sem_g0, sem_g1, sem_g2, sem_g3,
          sem_s0, sem_s1, sem_s2, sem_s3):
    wid = lax.axis_index("s") * NC + lax.axis_index("c")
    base = wid * CHUNK
    idx_lists = (idx0, idx1, idx2)
    pos_lists = (pos0, pos1, pos2)

    pltpu.sync_copy(ids_hbm.at[pl.ds(base, CHUNK)], ids_v)

    lane = lax.iota(jnp.int32, LANES)
    zero_lane = jnp.zeros((LANES,), jnp.int32)

    def compact(g, carry):
        o0, o1, o2, f0i, f0p, f1i, f1p, f2i, f2p = carry
        v = ids_v[pl.ds(g * LANES, LANES)]
        i = lane + g * LANES
        bb = lax.shift_right_logical(i * 1311, 16)
        pos = (i - bb * 50) * 1024 + (bb + (base // 50))
        m0 = v < VOCAB
        m2 = v >= VU
        r0 = _prefix_sum(jnp.where(m0, 1, 0), lane)
        r2 = _prefix_sum(jnp.where(m2, 1, 0), lane)
        r1 = (lane + 1) - r0 - r2
        n0 = r0[LANES - 1]
        n2 = r2[LANES - 1]
        cnts = (n0, LANES - n0 - n2, n2)
        firsts = []
        for r, o, cnt, fi, fp, idx_l, pos_l, off in (
                (r0, o0, cnts[0], f0i, f0p, idx0, pos0, 0),
                (r1, o1, cnts[1], f1i, f1p, idx1, pos1, VOCAB),
                (r2, o2, cnts[2], f2i, f2p, idx2, pos2, VU)):
            src = _compact_src(r, lane)
            cv = _permute(v, src) - off
            cp = _permute(pos, src)
            idx_l[pl.ds(o, LANES)] = cv
            pos_l[pl.ds(o, LANES)] = cp
            is_first = jnp.logical_and(o == 0, cnt > 0)
            fi = jnp.where(is_first, _permute(cv, zero_lane), fi)
            fp = jnp.where(is_first, _permute(cp, zero_lane), fp)
            firsts += [fi, fp]
        return (o0 + cnts[0], o1 + cnts[1], o2 + cnts[2], *firsts)

    z = jnp.int32(0)
    c0, c1, c2, f0i, f0p, f1i, f1p, f2i, f2p = lax.fori_loop(
        0, CHUNK // LANES, compact,
        (z, z, z, zero_lane, zero_lane, zero_lane, zero_lane, zero_lane,
         zero_lane))

    for c, fi, fp, idx_l, pos_l in ((c0, f0i, f0p, idx0, pos0),
                                    (c1, f1i, f1p, idx1, pos1),
                                    (c2, f2i, f2p, idx2, pos2)):
        for k in range(G // LANES):
            idx_l[pl.ds(c + k * LANES, LANES)] = fi
            pos_l[pl.ds(c + k * LANES, LANES)] = fp

    rows = (rows_a, rows_b, rows_c, rows_d)
    sg = (sem_g0, sem_g1, sem_g2, sem_g3)
    ss = (sem_s0, sem_s1, sem_s2, sem_s3)

    n0 = (c0 + (G - 1)) // G
    n1 = (c1 + (G - 1)) // G
    n2 = (c2 + (G - 1)) // G
    t1 = n0
    t2 = n0 + n1
    n_all = t2 + n2

    tables = ((wte_hbm, 0), (user_hbm, t1), (item_hbm, t2))
    bounds = ((0, t1), (t1, t2), (t2, n_all))

    def stage_and_gather(q, b):
        for t, (table, lo) in enumerate(tables):
            @pl.when(jnp.logical_and(q >= lo, q < bounds[t][1]))
            def _(t=t, table=table, lo=lo):
                qt = q - lo
                for k in range(G // LANES):
                    idx_stage.at[b][pl.ds(k * LANES, LANES)] = (
                        idx_lists[t][pl.ds(qt * G + k * LANES, LANES)])
                    pos_stage.at[b][pl.ds(k * LANES, LANES)] = (
                        pos_lists[t][pl.ds(qt * G + k * LANES, LANES)])
                pltpu.async_copy(table.at[idx_stage.at[b]], rows[b], sg[b])

    def wait_gather(b):
        pltpu.make_async_copy(wte_hbm.at[idx_stage.at[b]], rows[b],
                              sg[b]).wait()

    def drain_scatter(b):
        pltpu.make_async_copy(rows[b], out_hbm.at[pos_stage.at[b]],
                              ss[b]).wait()

    def ring_round(rd, _):
        for b in range(NB):
            q = rd * NB + b

            @pl.when(q < n_all)
            def _(q=q, b=b):
                @pl.when(q >= NB)
                def _():
                    drain_scatter(b)
                stage_and_gather(q, b)
        for b in range(NB):
            q = rd * NB + b

            @pl.when(q < n_all)
            def _(q=q, b=b):
                wait_gather(b)
                pltpu.async_copy(rows[b], out_hbm.at[pos_stage.at[b]], ss[b])
        return 0

    lax.fori_loop(0, (n_all + NB - 1) // NB, ring_round, 0)

    for b in range(NB):
        @pl.when(b < n_all)
        def _(b=b):
            drain_scatter(b)


@jax.jit
def _lookup(ids_flat, wte, user_embeddings, item_embeddings):
    mesh = plsc.VectorSubcoreMesh(core_axis_name="c", subcore_axis_name="s")
    return pl.kernel(
        _body,
        out_type=jax.ShapeDtypeStruct((TOTAL, N_EMBD), jnp.float32),
        mesh=mesh,
        scratch_types=[
            pltpu.VMEM((CHUNK,), jnp.int32),
            pltpu.VMEM((LIST,), jnp.int32),
            pltpu.VMEM((LIST,), jnp.int32),
            pltpu.VMEM((LIST,), jnp.int32),
            pltpu.VMEM((LIST,), jnp.int32),
            pltpu.VMEM((LIST,), jnp.int32),
            pltpu.VMEM((LIST,), jnp.int32),
            pltpu.VMEM((NB, G), jnp.int32),
            pltpu.VMEM((NB, G), jnp.int32),
            pltpu.VMEM((G, N_EMBD), jnp.float32),
            pltpu.VMEM((G, N_EMBD), jnp.float32),
            pltpu.VMEM((G, N_EMBD), jnp.float32),
            pltpu.VMEM((G, N_EMBD), jnp.float32),
            pltpu.SemaphoreType.DMA,
            pltpu.SemaphoreType.DMA,
            pltpu.SemaphoreType.DMA,
            pltpu.SemaphoreType.DMA,
            pltpu.SemaphoreType.DMA,
            pltpu.SemaphoreType.DMA,
            pltpu.SemaphoreType.DMA,
            pltpu.SemaphoreType.DMA,
        ],
    )(ids_flat, wte, user_embeddings, item_embeddings)


def kernel(input_ids, wte, user_embeddings, item_embeddings):
    ids_flat = input_ids.reshape(-1).astype(jnp.int32)
    out = _lookup(ids_flat, wte, user_embeddings, item_embeddings)
    return out.reshape(50, 1024, N_EMBD).transpose(1, 0, 2)
